# baseline (speedup 1.0000x reference)
import functools

import jax
import jax.numpy as jnp
from jax import lax
from jax.experimental import pallas as pl
from jax.experimental.pallas import tpu as pltpu
from jax.experimental.pallas import tpu_sc as plsc

NUM_CORES = 2
NUM_SUBCORES = 16
NW = NUM_CORES * NUM_SUBCORES
CHUNK = 8
DEPTH = 12
LAG = 6


@functools.lru_cache(maxsize=None)
def _build(N, D, V):
    NI = N // NW
    NCH = NI // CHUNK

    mesh = plsc.VectorSubcoreMesh(
        core_axis_name="c", subcore_axis_name="s",
        num_cores=NUM_CORES, num_subcores=NUM_SUBCORES,
    )

    @functools.partial(
        pl.kernel,
        out_type=jax.ShapeDtypeStruct((N, D), jnp.float32),
        mesh=mesh,
        compiler_params=pltpu.CompilerParams(needs_layout_passes=False),
        scratch_types=(
            [pltpu.VMEM((NI,), jnp.int32)]
            + [pltpu.VMEM((CHUNK, D), jnp.float32) for _ in range(DEPTH)]
            + [pltpu.SemaphoreType.DMA for _ in range(2 * DEPTH)]
        ),
    )
    def run(x_hbm, wt_hbm, out_hbm, xl_v, *bufs_and_sems):
        rows = list(bufs_and_sems[:DEPTH])
        sg = list(bufs_and_sems[DEPTH:2 * DEPTH])
        so = list(bufs_and_sems[2 * DEPTH:])
        c = lax.axis_index("c")
        s = lax.axis_index("s")
        wid = s * NUM_CORES + c
        i0 = wid * NI

        pltpu.sync_copy(x_hbm.at[pl.ds(i0, NI)], xl_v)

        def idx_slice(ch):
            return xl_v.at[pl.ds(ch * CHUNK, CHUNK)]

        def start_gather(ch, rows_ref, sem):
            pltpu.async_copy(wt_hbm.at[idx_slice(ch)], rows_ref, sem)

        def wait_gather(ch, rows_ref, sem):
            pltpu.make_async_copy(wt_hbm.at[idx_slice(ch)], rows_ref, sem).wait()

        def out_slice(ch):
            return out_hbm.at[pl.ds(i0 + ch * CHUNK, CHUNK), :]

        def start_out(rows_ref, ch, sem):
            pltpu.async_copy(rows_ref, out_slice(ch), sem)

        def wait_out(rows_ref, ch, sem):
            pltpu.make_async_copy(rows_ref, out_slice(ch), sem).wait()

        for ch in range(NCH):
            b = ch % DEPTH
            if ch >= DEPTH:
                wait_out(rows[b], ch - DEPTH, so[b])
            start_gather(ch, rows[b], sg[b])
            if ch >= LAG:
                b2 = (ch - LAG) % DEPTH
                wait_gather(ch - LAG, rows[b2], sg[b2])
                start_out(rows[b2], ch - LAG, so[b2])
        for ch in range(NCH - LAG, NCH):
            b = ch % DEPTH
            wait_gather(ch, rows[b], sg[b])
            start_out(rows[b], ch, so[b])
        for ch in range(NCH - DEPTH, NCH):
            b = ch % DEPTH
            wait_out(rows[b], ch, so[b])

    return run


def kernel(x, W_E):
    B, S = x.shape
    D, V = W_E.shape
    N = B * S
    x_flat = x.reshape(N).astype(jnp.int32)
    wt = W_E.T
    out = _build(N, D, V)(x_flat, wt)
    return out.reshape(B, S, D)


# speedup vs baseline: 1.0310x; 1.0310x over previous
import functools

import jax
import jax.numpy as jnp
from jax import lax
from jax.experimental import pallas as pl
from jax.experimental.pallas import tpu as pltpu
from jax.experimental.pallas import tpu_sc as plsc

NUM_CORES = 2
NUM_SUBCORES = 16
NW = NUM_CORES * NUM_SUBCORES
CHUNK = 16
DEPTH = 7
LAG = 4


@functools.lru_cache(maxsize=None)
def _build(N, D, V):
    NI = N // NW
    NCH = NI // CHUNK

    mesh = plsc.VectorSubcoreMesh(
        core_axis_name="c", subcore_axis_name="s",
        num_cores=NUM_CORES, num_subcores=NUM_SUBCORES,
    )

    @functools.partial(
        pl.kernel,
        out_type=jax.ShapeDtypeStruct((N, D), jnp.float32),
        mesh=mesh,
        compiler_params=pltpu.CompilerParams(needs_layout_passes=False),
        scratch_types=(
            [pltpu.VMEM((NI,), jnp.int32)]
            + [pltpu.VMEM((CHUNK, D), jnp.float32) for _ in range(DEPTH)]
            + [pltpu.SemaphoreType.DMA for _ in range(2 * DEPTH)]
        ),
    )
    def run(x_hbm, wt_hbm, out_hbm, xl_v, *bufs_and_sems):
        rows = list(bufs_and_sems[:DEPTH])
        sg = list(bufs_and_sems[DEPTH:2 * DEPTH])
        so = list(bufs_and_sems[2 * DEPTH:])
        c = lax.axis_index("c")
        s = lax.axis_index("s")
        wid = s * NUM_CORES + c
        i0 = wid * NI

        pltpu.sync_copy(x_hbm.at[pl.ds(i0, NI)], xl_v)

        def idx_slice(ch):
            return xl_v.at[pl.ds(ch * CHUNK, CHUNK)]

        def start_gather(ch, rows_ref, sem):
            pltpu.async_copy(wt_hbm.at[idx_slice(ch)], rows_ref, sem)

        def wait_gather(ch, rows_ref, sem):
            pltpu.make_async_copy(wt_hbm.at[idx_slice(ch)], rows_ref, sem).wait()

        def out_slice(ch):
            return out_hbm.at[pl.ds(i0 + ch * CHUNK, CHUNK), :]

        def start_out(rows_ref, ch, sem):
            pltpu.async_copy(rows_ref, out_slice(ch), sem)

        def wait_out(rows_ref, ch, sem):
            pltpu.make_async_copy(rows_ref, out_slice(ch), sem).wait()

        for ch in range(NCH):
            b = ch % DEPTH
            if ch >= DEPTH:
                wait_out(rows[b], ch - DEPTH, so[b])
            start_gather(ch, rows[b], sg[b])
            if ch >= LAG:
                b2 = (ch - LAG) % DEPTH
                wait_gather(ch - LAG, rows[b2], sg[b2])
                start_out(rows[b2], ch - LAG, so[b2])
        for ch in range(NCH - LAG, NCH):
            b = ch % DEPTH
            wait_gather(ch, rows[b], sg[b])
            start_out(rows[b], ch, so[b])
        for ch in range(NCH - DEPTH, NCH):
            b = ch % DEPTH
            wait_out(rows[b], ch, so[b])

    return run


def kernel(x, W_E):
    B, S = x.shape
    D, V = W_E.shape
    N = B * S
    x_flat = x.reshape(N).astype(jnp.int32)
    wt = W_E.T
    out = _build(N, D, V)(x_flat, wt)
    return out.reshape(B, S, D)
